# baseline (speedup 1.0000x reference)
import functools

import jax
import jax.numpy as jnp
from jax import lax
from jax.experimental import pallas as pl
from jax.experimental.pallas import tpu as pltpu
from jax.experimental.pallas import tpu_sc as plsc

G = 8
K = 1024
D = 64
N = 9216
NH = 2
N2 = N // NH
BN = 512
NBLK = N2 // BN

NC = 2
NS = 16
NW = NC * NS
B2 = N2 * G
B_PER_W = B2 // NW
CHUNK = 576
NCHUNK = B_PER_W // CHUNK


def _tc_body(x_ref, cb_ref, idx_ref, loss_ref, w2_ref, cbn2_ref, iota_ref):
    @pl.when(pl.program_id(0) == 0)
    def _init():
        for g in range(G):
            cbg = cb_ref[g]
            cbn2_ref[g] = -2.0 * cbg
            w2_ref[g] = jnp.sum(cbg * cbg, axis=-1).reshape(1, K)
        iota_ref[...] = lax.broadcasted_iota(
            jnp.int32, (8, K), 1).astype(jnp.float32)

    acc = jnp.float32(0.0)
    for g in range(G):
        xg = x_ref[:, g * D:(g + 1) * D]
        xw2 = lax.dot_general(xg, cbn2_ref[g], (((1,), (1,)), ((), ())),
                              preferred_element_type=jnp.float32)
        x2 = jnp.sum(xg * xg, axis=-1, keepdims=True)
        dist = (x2 + xw2) + w2_ref[g]
        m = jnp.min(dist, axis=-1, keepdims=True)
        iota = iota_ref[0:1, :]
        idxf = jnp.min(jnp.where(dist == m, iota, jnp.float32(K)),
                       axis=-1, keepdims=True)
        idx_ref[:, g:g + 1] = idxf.astype(jnp.int32) + g * K
        acc = acc + jnp.sum(m)
    loss_ref[...] = acc.reshape(1, 1, 1)


def _tc_distance_argmin(flat, codebooks, half):
    return pl.pallas_call(
        _tc_body,
        grid=(NBLK,),
        in_specs=[
            pl.BlockSpec((BN, G * D), lambda i, h=half: (i + h * NBLK, 0)),
            pl.BlockSpec((G, K, D), lambda i: (0, 0, 0)),
        ],
        out_specs=[
            pl.BlockSpec((BN, G), lambda i: (i, 0)),
            pl.BlockSpec((1, 1, 1), lambda i: (i, 0, 0)),
        ],
        out_shape=[
            jax.ShapeDtypeStruct((N2, G), jnp.int32),
            jax.ShapeDtypeStruct((NBLK, 1, 1), jnp.float32),
        ],
        scratch_shapes=[
            pltpu.VMEM((G, 1, K), jnp.float32),
            pltpu.VMEM((G, K, D), jnp.float32),
            pltpu.VMEM((8, K), jnp.float32),
        ],
    )(flat, codebooks)


def _sc_decode(table, idx_flat):
    mesh = plsc.VectorSubcoreMesh(core_axis_name="c", subcore_axis_name="s")

    @functools.partial(
        pl.kernel, mesh=mesh,
        out_type=jax.ShapeDtypeStruct((B2, 2 * D), jnp.float32),
        scratch_types=[
            pltpu.VMEM((CHUNK,), jnp.int32),
            pltpu.VMEM((CHUNK, 2 * D), jnp.float32),
            pltpu.SemaphoreType.DMA,
        ],
    )
    def k(table_hbm, idx_hbm, out_hbm, idx_v, rows_v, sem):
        wid = lax.axis_index("s") * NC + lax.axis_index("c")
        base = wid * B_PER_W
        for c in range(NCHUNK):
            off = base + c * CHUNK
            pltpu.sync_copy(idx_hbm.at[pl.ds(off, CHUNK)], idx_v)
            pltpu.async_copy(table_hbm.at[idx_v], rows_v, sem).wait()
            pltpu.sync_copy(rows_v, out_hbm.at[pl.ds(off, CHUNK)])

    return k(table, idx_flat)


def kernel(latents, codebooks):
    flat = latents.reshape(N, G * D)
    cb_flat = codebooks.reshape(G * K, D)
    table = jnp.concatenate([cb_flat, cb_flat], axis=1)
    quants, losses = [], []
    for h in range(NH):
        idx_h, loss_h = _tc_distance_argmin(flat, codebooks, h)
        quants.append(_sc_decode(table, idx_h.reshape(B2))[:, :D])
        losses.append(jnp.sum(loss_h))
    quantized = jnp.concatenate(quants, axis=0).reshape(latents.shape)
    vq_loss = (losses[0] + losses[1]) * (1.25 / (N * G * D))
    return (quantized, vq_loss)


# speedup vs baseline: 1.1219x; 1.1219x over previous
import functools

import jax
import jax.numpy as jnp
from jax import lax
from jax.experimental import pallas as pl
from jax.experimental.pallas import tpu as pltpu
from jax.experimental.pallas import tpu_sc as plsc

G = 8
K = 1024
D = 64
N = 9216
BN = 512
NBLK = N // BN

NC = 2
NS = 16
NW = NC * NS
B = N * G
B_PER_W = B // NW
CHUNK = 576
NCHUNK = B_PER_W // CHUNK


def _tc_body(x_ref, cb_ref, idx_ref, loss_ref, w2_ref, cbn2_ref, iota_ref):
    @pl.when(pl.program_id(0) == 0)
    def _init():
        for g in range(G):
            cbg = cb_ref[g]
            cbn2_ref[g] = -2.0 * cbg
            w2_ref[g] = jnp.sum(cbg * cbg, axis=-1).reshape(1, K)
        iota_ref[...] = lax.broadcasted_iota(
            jnp.int32, (8, K), 1).astype(jnp.float32)

    acc = jnp.float32(0.0)
    for g in range(G):
        xg = x_ref[:, g * D:(g + 1) * D]
        xw2 = lax.dot_general(xg, cbn2_ref[g], (((1,), (1,)), ((), ())),
                              preferred_element_type=jnp.float32)
        x2 = jnp.sum(xg * xg, axis=-1, keepdims=True)
        dist = (x2 + xw2) + w2_ref[g]
        m = jnp.min(dist, axis=-1, keepdims=True)
        iota = iota_ref[0:1, :]
        idxf = jnp.min(jnp.where(dist == m, iota, jnp.float32(K)),
                       axis=-1, keepdims=True)
        idx_ref[:, g:g + 1] = idxf.astype(jnp.int32) + g * K
        acc = acc + jnp.sum(m)
    loss_ref[...] = acc.reshape(1, 1, 1)


def _tc_distance_argmin(flat, codebooks):
    return pl.pallas_call(
        _tc_body,
        grid=(NBLK,),
        in_specs=[
            pl.BlockSpec((BN, G * D), lambda i: (i, 0)),
            pl.BlockSpec((G, K, D), lambda i: (0, 0, 0)),
        ],
        out_specs=[
            pl.BlockSpec((BN, G), lambda i: (i, 0)),
            pl.BlockSpec((1, 1, 1), lambda i: (i, 0, 0)),
        ],
        out_shape=[
            jax.ShapeDtypeStruct((N, G), jnp.int32),
            jax.ShapeDtypeStruct((NBLK, 1, 1), jnp.float32),
        ],
        scratch_shapes=[
            pltpu.VMEM((G, 1, K), jnp.float32),
            pltpu.VMEM((G, K, D), jnp.float32),
            pltpu.VMEM((8, K), jnp.float32),
        ],
    )(flat, codebooks)


def _sc_decode(table, idx_flat):
    mesh = plsc.VectorSubcoreMesh(core_axis_name="c", subcore_axis_name="s")

    @functools.partial(
        pl.kernel, mesh=mesh,
        out_type=jax.ShapeDtypeStruct((B, 2 * D), jnp.float32),
        scratch_types=[
            pltpu.VMEM((CHUNK,), jnp.int32),
            pltpu.VMEM((CHUNK, 2 * D), jnp.float32),
            pltpu.SemaphoreType.DMA,
        ],
    )
    def k(table_hbm, idx_hbm, out_hbm, idx_v, rows_v, sem):
        wid = lax.axis_index("s") * NC + lax.axis_index("c")
        base = wid * B_PER_W
        for c in range(NCHUNK):
            off = base + c * CHUNK
            pltpu.sync_copy(idx_hbm.at[pl.ds(off, CHUNK)], idx_v)
            pltpu.async_copy(table_hbm.at[idx_v], rows_v, sem).wait()
            pltpu.sync_copy(rows_v, out_hbm.at[pl.ds(off, CHUNK)])

    return k(table, idx_flat)


def _compact_body(in_ref, out_ref):
    x = in_ref[...].reshape(BN, G, 2 * D)
    out_ref[...] = x[:, :, :D].reshape(BN, G * D)


def _compact(quant_pad):
    return pl.pallas_call(
        _compact_body,
        grid=(NBLK,),
        in_specs=[pl.BlockSpec((BN * G, 2 * D), lambda i: (i, 0))],
        out_specs=pl.BlockSpec((BN, G * D), lambda i: (i, 0)),
        out_shape=jax.ShapeDtypeStruct((N, G * D), jnp.float32),
    )(quant_pad)


def kernel(latents, codebooks):
    flat = latents.reshape(N, G * D)
    idx_all, loss_parts = _tc_distance_argmin(flat, codebooks)
    cb_flat = codebooks.reshape(G * K, D)
    table = jnp.concatenate([cb_flat, cb_flat], axis=1)
    quant_pad = _sc_decode(table, idx_all.reshape(B))
    quantized = _compact(quant_pad).reshape(latents.shape)
    vq_loss = jnp.sum(loss_parts) * (1.25 / (N * G * D))
    return (quantized, vq_loss)


# speedup vs baseline: 1.1906x; 1.0612x over previous
import functools

import jax
import jax.numpy as jnp
from jax import lax
from jax.experimental import pallas as pl
from jax.experimental.pallas import tpu as pltpu
from jax.experimental.pallas import tpu_sc as plsc

G = 8
K = 1024
D = 64
N = 9216
BN = 1024
NBLK = N // BN
BC = 512
NBLKC = N // BC

NC = 2
NS = 16
NW = NC * NS
B = N * G
B_PER_W = B // NW
CHUNK = 576
NCHUNK = B_PER_W // CHUNK


def _tc_body(x_ref, cb_ref, idx_ref, loss_ref, table_ref,
             w2_ref, cbn2_ref, iota_ref):
    @pl.when(pl.program_id(0) == 0)
    def _init():
        for g in range(G):
            cbg = cb_ref[g]
            cbn2_ref[g] = -2.0 * cbg
            w2_ref[g] = jnp.sum(cbg * cbg, axis=-1).reshape(1, K)
        iota_ref[...] = lax.broadcasted_iota(
            jnp.int32, (8, K), 1).astype(jnp.float32)
        cb2d = cb_ref[...].reshape(G * K, D)
        table_ref[:, :D] = cb2d
        table_ref[:, D:] = cb2d

    acc = jnp.float32(0.0)
    for g in range(G):
        xg = x_ref[:, g * D:(g + 1) * D]
        xw2 = lax.dot_general(xg, cbn2_ref[g], (((1,), (1,)), ((), ())),
                              preferred_element_type=jnp.float32)
        x2 = jnp.sum(xg * xg, axis=-1, keepdims=True)
        dist = (x2 + xw2) + w2_ref[g]
        m = jnp.min(dist, axis=-1, keepdims=True)
        iota = iota_ref[0:1, :]
        idxf = jnp.min(jnp.where(dist == m, iota, jnp.float32(K)),
                       axis=-1, keepdims=True)
        idx_ref[:, g:g + 1] = idxf.astype(jnp.int32) + g * K
        acc = acc + jnp.sum(m)
    loss_ref[...] = acc.reshape(1, 1, 1)


def _tc_distance_argmin(flat, codebooks):
    return pl.pallas_call(
        _tc_body,
        grid=(NBLK,),
        in_specs=[
            pl.BlockSpec((BN, G * D), lambda i: (i, 0)),
            pl.BlockSpec((G, K, D), lambda i: (0, 0, 0)),
        ],
        out_specs=[
            pl.BlockSpec((BN, G), lambda i: (i, 0)),
            pl.BlockSpec((1, 1, 1), lambda i: (i, 0, 0)),
            pl.BlockSpec((G * K, 2 * D), lambda i: (0, 0)),
        ],
        out_shape=[
            jax.ShapeDtypeStruct((N, G), jnp.int32),
            jax.ShapeDtypeStruct((NBLK, 1, 1), jnp.float32),
            jax.ShapeDtypeStruct((G * K, 2 * D), jnp.float32),
        ],
        scratch_shapes=[
            pltpu.VMEM((G, 1, K), jnp.float32),
            pltpu.VMEM((G, K, D), jnp.float32),
            pltpu.VMEM((8, K), jnp.float32),
        ],
    )(flat, codebooks)


def _sc_decode(table, idx_flat):
    mesh = plsc.VectorSubcoreMesh(core_axis_name="c", subcore_axis_name="s")

    @functools.partial(
        pl.kernel, mesh=mesh,
        out_type=jax.ShapeDtypeStruct((B, 2 * D), jnp.float32),
        scratch_types=[
            pltpu.VMEM((CHUNK,), jnp.int32),
            pltpu.VMEM((CHUNK, 2 * D), jnp.float32),
            pltpu.SemaphoreType.DMA,
        ],
    )
    def k(table_hbm, idx_hbm, out_hbm, idx_v, rows_v, sem):
        wid = lax.axis_index("s") * NC + lax.axis_index("c")
        base = wid * B_PER_W
        for c in range(NCHUNK):
            off = base + c * CHUNK
            pltpu.sync_copy(idx_hbm.at[pl.ds(off, CHUNK)], idx_v)
            pltpu.async_copy(table_hbm.at[idx_v], rows_v, sem).wait()
            pltpu.sync_copy(rows_v, out_hbm.at[pl.ds(off, CHUNK)])

    return k(table, idx_flat)


def _compact_body(in_ref, out_ref):
    x = in_ref[...].reshape(BC, G, 2 * D)
    out_ref[...] = x[:, :, :D].reshape(BC, G * D)


def _compact(quant_pad):
    return pl.pallas_call(
        _compact_body,
        grid=(NBLKC,),
        in_specs=[pl.BlockSpec((BC * G, 2 * D), lambda i: (i, 0))],
        out_specs=pl.BlockSpec((BC, G * D), lambda i: (i, 0)),
        out_shape=jax.ShapeDtypeStruct((N, G * D), jnp.float32),
    )(quant_pad)


def kernel(latents, codebooks):
    flat = latents.reshape(N, G * D)
    idx_all, loss_parts, table = _tc_distance_argmin(flat, codebooks)
    quant_pad = _sc_decode(table, idx_all.reshape(B))
    quantized = _compact(quant_pad).reshape(latents.shape)
    vq_loss = jnp.sum(loss_parts) * (1.25 / (N * G * D))
    return (quantized, vq_loss)
